# Initial kernel scaffold; baseline (speedup 1.0000x reference)
import functools

import jax
import jax.numpy as jnp
from jax import lax
from jax.experimental import pallas as pl
from jax.experimental.pallas import tpu as pltpu
from jax.experimental.pallas import tpu_sc as plsc

E = 8
D = 768
H = 2048
T = 2048
COEF = 0.01

LANES = 128
GBLK = 256
NGB = T // GBLK
BLK = 256
MAX_TILES = 16
PADDED = MAX_TILES * BLK
NC, NS = 2, 16
NW = NC * NS
CHUNK = T // NW


def _gate_body(x_ref, gw_ref, gb_ref, choice_ref, poswi_ref, counts_ref,
               loss_ref, carry_ref):
    i = pl.program_id(0)

    @pl.when(i == 0)
    def _():
        carry_ref[...] = jnp.zeros((1, LANES), jnp.float32)

    xb = x_ref[...]
    logits = jnp.dot(xb, gw_ref[...],
                     preferred_element_type=jnp.float32) + gb_ref[...]
    lane = lax.broadcasted_iota(jnp.int32, (GBLK, LANES), 1)
    neg = jnp.full((GBLK, LANES), -jnp.inf, jnp.float32)
    lm = jnp.where(lane < E, logits, neg)
    mx = jnp.max(lm, axis=1, keepdims=True)
    ex = jnp.exp(lm - mx)
    probs = ex / jnp.sum(ex, axis=1, keepdims=True)
    pmax = jnp.max(probs, axis=1, keepdims=True)
    first = jnp.where(probs == pmax, lane, LANES)
    choice = jnp.min(first, axis=1)

    onehot = (lane == choice[:, None]).astype(jnp.float32)
    r = lax.broadcasted_iota(jnp.int32, (GBLK, GBLK), 0)
    c = lax.broadcasted_iota(jnp.int32, (GBLK, GBLK), 1)
    tri = (c < r).astype(jnp.float32)
    rank = jnp.dot(tri, onehot,
                   preferred_element_type=jnp.float32) + carry_ref[...]
    poswi = jnp.sum(onehot * rank, axis=1)

    choice_ref[...] = choice.reshape(1, 1, GBLK)
    poswi_ref[...] = poswi.astype(jnp.int32).reshape(1, 1, GBLK)

    totals = carry_ref[...] + jnp.sum(onehot, axis=0, keepdims=True)
    carry_ref[...] = totals

    @pl.when(i == NGB - 1)
    def _():
        counts_ref[...] = totals.astype(jnp.int32)
        p = totals / float(T)
        loss = -jnp.sum(p * jnp.log(p + 1e-10)) * COEF
        loss_ref[...] = jnp.full((1, LANES), loss, jnp.float32)


def _gate_call(xf, gw_p, gb_p):
    return pl.pallas_call(
        _gate_body,
        grid=(NGB,),
        in_specs=[
            pl.BlockSpec((GBLK, D), lambda i: (i, 0)),
            pl.BlockSpec((D, LANES), lambda i: (0, 0)),
            pl.BlockSpec((1, LANES), lambda i: (0, 0)),
        ],
        out_specs=[
            pl.BlockSpec((1, 1, GBLK), lambda i: (i, 0, 0)),
            pl.BlockSpec((1, 1, GBLK), lambda i: (i, 0, 0)),
            pl.BlockSpec((1, LANES), lambda i: (0, 0)),
            pl.BlockSpec((1, LANES), lambda i: (0, 0)),
        ],
        out_shape=[
            jax.ShapeDtypeStruct((NGB, 1, GBLK), jnp.int32),
            jax.ShapeDtypeStruct((NGB, 1, GBLK), jnp.int32),
            jax.ShapeDtypeStruct((1, LANES), jnp.int32),
            jax.ShapeDtypeStruct((1, LANES), jnp.float32),
        ],
        scratch_shapes=[pltpu.VMEM((1, LANES), jnp.float32)],
    )(xf, gw_p, gb_p)


def _dispatch_body(choice_hbm, poswi_hbm, off_hbm, x_hbm, xs_hbm, pos_hbm,
                   choice_v, poswi_v, off_v, pos_v, rows_v, sem):
    wid = lax.axis_index("s") * NC + lax.axis_index("c")
    base = wid * CHUNK
    pltpu.sync_copy(choice_hbm.at[pl.ds(base, CHUNK)], choice_v)
    pltpu.sync_copy(poswi_hbm.at[pl.ds(base, CHUNK)], poswi_v)
    pltpu.sync_copy(off_hbm, off_v)
    for j in range(CHUNK // 16):
        cv = choice_v[pl.ds(j * 16, 16)]
        pw = poswi_v[pl.ds(j * 16, 16)]
        ov = plsc.load_gather(off_v, [cv])
        pos_v[pl.ds(j * 16, 16)] = ov + pw
    pltpu.sync_copy(x_hbm.at[pl.ds(base, CHUNK)], rows_v)
    pltpu.async_copy(rows_v, xs_hbm.at[pos_v], sem).wait()
    pltpu.sync_copy(pos_v, pos_hbm.at[pl.ds(base, CHUNK)])


def _dispatch_call(choice, poswi, off16, xf):
    mesh = plsc.VectorSubcoreMesh(core_axis_name="c", subcore_axis_name="s")
    fn = functools.partial(
        pl.kernel,
        mesh=mesh,
        out_type=[
            jax.ShapeDtypeStruct((PADDED, D), jnp.float32),
            jax.ShapeDtypeStruct((T,), jnp.int32),
        ],
        scratch_types=[
            pltpu.VMEM((CHUNK,), jnp.int32),
            pltpu.VMEM((CHUNK,), jnp.int32),
            pltpu.VMEM((16,), jnp.int32),
            pltpu.VMEM((CHUNK,), jnp.int32),
            pltpu.VMEM((CHUNK, D), jnp.float32),
            pltpu.SemaphoreType.DMA,
        ],
    )(_dispatch_body)
    return fn(choice, poswi, off16, xf)


def _ffn_body(te_ref, tv_ref, xs_ref, w1_ref, b1_ref, w2_ref, b2_ref,
              pw_ref, pb_ref, out_ref):
    j = pl.program_id(0)

    @pl.when(tv_ref[j] == 1)
    def _():
        xb = xs_ref[...]
        h = jnp.dot(xb, w1_ref[0],
                    preferred_element_type=jnp.float32) + b1_ref[...]
        h = jnp.maximum(h, 0.0)
        y = jnp.dot(h, w2_ref[0],
                    preferred_element_type=jnp.float32) + b2_ref[...]
        out_ref[...] = jnp.dot(
            y, pw_ref[...], preferred_element_type=jnp.float32) + pb_ref[...]


def _ffn_call(te, tv, xs, W1, b1, W2, b2, proj_W, proj_b2d):
    grid_spec = pltpu.PrefetchScalarGridSpec(
        num_scalar_prefetch=2,
        grid=(MAX_TILES,),
        in_specs=[
            pl.BlockSpec((BLK, D), lambda j, te, tv: (j, 0)),
            pl.BlockSpec((1, D, H), lambda j, te, tv: (te[j], 0, 0)),
            pl.BlockSpec((1, H), lambda j, te, tv: (te[j], 0)),
            pl.BlockSpec((1, H, D), lambda j, te, tv: (te[j], 0, 0)),
            pl.BlockSpec((1, D), lambda j, te, tv: (te[j], 0)),
            pl.BlockSpec((D, D), lambda j, te, tv: (0, 0)),
            pl.BlockSpec((1, D), lambda j, te, tv: (0, 0)),
        ],
        out_specs=pl.BlockSpec((BLK, D), lambda j, te, tv: (j, 0)),
    )
    return pl.pallas_call(
        _ffn_body,
        grid_spec=grid_spec,
        out_shape=jax.ShapeDtypeStruct((PADDED, D), jnp.float32),
    )(te, tv, xs, W1, b1, W2, b2, proj_W, proj_b2d)


def _combine_body(pos_hbm, ys_hbm, out_hbm, pos_v, rows_v, sem):
    wid = lax.axis_index("s") * NC + lax.axis_index("c")
    base = wid * CHUNK
    pltpu.sync_copy(pos_hbm.at[pl.ds(base, CHUNK)], pos_v)
    pltpu.async_copy(ys_hbm.at[pos_v], rows_v, sem).wait()
    pltpu.sync_copy(rows_v, out_hbm.at[pl.ds(base, CHUNK)])


def _combine_call(pos, ys):
    mesh = plsc.VectorSubcoreMesh(core_axis_name="c", subcore_axis_name="s")
    fn = functools.partial(
        pl.kernel,
        mesh=mesh,
        out_type=jax.ShapeDtypeStruct((T, D), jnp.float32),
        scratch_types=[
            pltpu.VMEM((CHUNK,), jnp.int32),
            pltpu.VMEM((CHUNK, D), jnp.float32),
            pltpu.SemaphoreType.DMA,
        ],
    )(_combine_body)
    return fn(pos, ys)


def kernel(x, gate_W, gate_b, W1, b1, W2, b2, proj_W, proj_b):
    bs, seq_len, d_model = x.shape
    xf = x.reshape(T, D)
    gw_p = jnp.zeros((D, LANES), jnp.float32).at[:, :E].set(gate_W)
    gb_p = jnp.zeros((1, LANES), jnp.float32).at[0, :E].set(gate_b)

    choice3, poswi3, counts_o, loss_o = _gate_call(xf, gw_p, gb_p)
    choice = choice3.reshape(T)
    poswi = poswi3.reshape(T)
    counts = counts_o[0, :E]

    padded = ((counts + BLK - 1) // BLK) * BLK
    offs = jnp.concatenate(
        [jnp.zeros((1,), jnp.int32), jnp.cumsum(padded).astype(jnp.int32)])
    total = offs[E]
    off16 = jnp.zeros((16,), jnp.int32).at[:E].set(offs[:E])
    jB = jnp.arange(MAX_TILES, dtype=jnp.int32) * BLK
    te = (jnp.sum((offs[None, :E] <= jB[:, None]).astype(jnp.int32),
                  axis=1) - 1).astype(jnp.int32)
    tv = (jB < total).astype(jnp.int32)

    xs, pos = _dispatch_call(choice, poswi, off16, xf)
    ys = _ffn_call(te, tv, xs, W1, b1, W2, b2, proj_W,
                   proj_b.reshape(1, D))
    out = _combine_call(pos, ys)

    loss = loss_o[0, 0].reshape(())
    return out.reshape(bs, seq_len, d_model), loss


# speedup vs baseline: 3.0031x; 3.0031x over previous
import functools

import jax
import jax.numpy as jnp
from jax import lax
from jax.experimental import pallas as pl
from jax.experimental.pallas import tpu as pltpu
from jax.experimental.pallas import tpu_sc as plsc

E = 8
D = 768
H = 2048
T = 2048
COEF = 0.01

LANES = 128
GBLK = 256
NGB = T // GBLK
BLK = 256
MAX_TILES = 16
PADDED = MAX_TILES * BLK
NC, NS = 2, 16
NW = NC * NS
CHUNK = T // NW


def _gate_body(x_ref, gw_ref, gb_ref, pos_ref, counts_ref, loss_ref,
               carry_ref, choice_s, poswi_s):
    i = pl.program_id(0)

    @pl.when(i == 0)
    def _():
        carry_ref[...] = jnp.zeros((1, LANES), jnp.float32)

    @pl.when(i < NGB)
    def _():
        xb = x_ref[...]
        logits = jnp.dot(xb, gw_ref[...],
                         preferred_element_type=jnp.float32) + gb_ref[...]
        lane = lax.broadcasted_iota(jnp.int32, (GBLK, LANES), 1)
        neg = jnp.full((GBLK, LANES), -jnp.inf, jnp.float32)
        lm = jnp.where(lane < E, logits, neg)
        mx = jnp.max(lm, axis=1, keepdims=True)
        ex = jnp.exp(lm - mx)
        probs = ex / jnp.sum(ex, axis=1, keepdims=True)
        pmax = jnp.max(probs, axis=1, keepdims=True)
        first = jnp.where(probs == pmax, lane, LANES)
        choice = jnp.min(first, axis=1)

        onehot = (lane == choice[:, None]).astype(jnp.float32)
        r = lax.broadcasted_iota(jnp.int32, (GBLK, GBLK), 0)
        c = lax.broadcasted_iota(jnp.int32, (GBLK, GBLK), 1)
        tri = (c < r).astype(jnp.float32)
        rank = jnp.dot(tri, onehot,
                       preferred_element_type=jnp.float32) + carry_ref[...]
        poswi = jnp.sum(onehot * rank, axis=1)

        choice_s[pl.ds(i, 1), :] = choice.reshape(1, GBLK)
        poswi_s[pl.ds(i, 1), :] = poswi.astype(jnp.int32).reshape(1, GBLK)
        carry_ref[...] = carry_ref[...] + jnp.sum(onehot, axis=0,
                                                  keepdims=True)

    @pl.when(i == NGB)
    def _():
        totals = carry_ref[...]
        counts_ref[...] = totals.astype(jnp.int32)
        p = totals / float(T)
        loss = -jnp.sum(p * jnp.log(p + 1e-10)) * COEF
        loss_ref[...] = jnp.full((1, LANES), loss, jnp.float32)
        ch = choice_s[...]
        pw = poswi_s[...]
        acc = jnp.zeros((NGB, GBLK), jnp.int32)
        off = jnp.int32(0)
        for e in range(E):
            acc = jnp.where(ch == e, off + pw, acc)
            cnt = totals[0, e].astype(jnp.int32)
            off = off + ((cnt + BLK - 1) // BLK) * BLK
        pos_ref[...] = acc.reshape(NGB, 1, GBLK)


def _gate_call(xf, gw_p, gb_p):
    return pl.pallas_call(
        _gate_body,
        grid=(NGB + 1,),
        in_specs=[
            pl.BlockSpec((GBLK, D), lambda i: (jnp.minimum(i, NGB - 1), 0)),
            pl.BlockSpec((D, LANES), lambda i: (0, 0)),
            pl.BlockSpec((1, LANES), lambda i: (0, 0)),
        ],
        out_specs=[
            pl.BlockSpec((NGB, 1, GBLK), lambda i: (0, 0, 0)),
            pl.BlockSpec((1, LANES), lambda i: (0, 0)),
            pl.BlockSpec((1, LANES), lambda i: (0, 0)),
        ],
        out_shape=[
            jax.ShapeDtypeStruct((NGB, 1, GBLK), jnp.int32),
            jax.ShapeDtypeStruct((1, LANES), jnp.int32),
            jax.ShapeDtypeStruct((1, LANES), jnp.float32),
        ],
        scratch_shapes=[
            pltpu.VMEM((1, LANES), jnp.float32),
            pltpu.VMEM((NGB, GBLK), jnp.int32),
            pltpu.VMEM((NGB, GBLK), jnp.int32),
        ],
    )(xf, gw_p, gb_p)


def _dispatch_body(pos_hbm, x_hbm, xs_hbm, pos_v, rows_v, sem):
    wid = lax.axis_index("s") * NC + lax.axis_index("c")
    base = wid * CHUNK
    pltpu.sync_copy(pos_hbm.at[pl.ds(base, CHUNK)], pos_v)
    pltpu.sync_copy(x_hbm.at[pl.ds(base, CHUNK)], rows_v)
    pltpu.async_copy(rows_v, xs_hbm.at[pos_v], sem).wait()


def _dispatch_call(pos, xf):
    mesh = plsc.VectorSubcoreMesh(core_axis_name="c", subcore_axis_name="s")
    fn = functools.partial(
        pl.kernel,
        mesh=mesh,
        out_type=jax.ShapeDtypeStruct((PADDED, D), jnp.float32),
        scratch_types=[
            pltpu.VMEM((CHUNK,), jnp.int32),
            pltpu.VMEM((CHUNK, D), jnp.float32),
            pltpu.SemaphoreType.DMA,
        ],
    )(_dispatch_body)
    return fn(pos, xf)


def _ffn_body(te_ref, tv_ref, xs_ref, w1_ref, b1_ref, w2_ref, b2_ref,
              pw_ref, pb_ref, out_ref):
    j = pl.program_id(0)

    @pl.when(tv_ref[j] == 1)
    def _():
        xb = xs_ref[...]
        h = jnp.dot(xb, w1_ref[0],
                    preferred_element_type=jnp.float32) + b1_ref[0]
        h = jnp.maximum(h, 0.0)
        y = jnp.dot(h, w2_ref[0],
                    preferred_element_type=jnp.float32) + b2_ref[0]
        out_ref[...] = jnp.dot(
            y, pw_ref[...], preferred_element_type=jnp.float32) + pb_ref[...]


def _ffn_call(te, tv, xs, W1, b1, W2, b2, proj_W, proj_b2d):
    grid_spec = pltpu.PrefetchScalarGridSpec(
        num_scalar_prefetch=2,
        grid=(MAX_TILES,),
        in_specs=[
            pl.BlockSpec((BLK, D), lambda j, te, tv: (j, 0)),
            pl.BlockSpec((1, D, H), lambda j, te, tv: (te[j], 0, 0)),
            pl.BlockSpec((1, 1, H), lambda j, te, tv: (te[j], 0, 0)),
            pl.BlockSpec((1, H, D), lambda j, te, tv: (te[j], 0, 0)),
            pl.BlockSpec((1, 1, D), lambda j, te, tv: (te[j], 0, 0)),
            pl.BlockSpec((D, D), lambda j, te, tv: (0, 0)),
            pl.BlockSpec((1, D), lambda j, te, tv: (0, 0)),
        ],
        out_specs=pl.BlockSpec((BLK, D), lambda j, te, tv: (j, 0)),
    )
    return pl.pallas_call(
        _ffn_body,
        grid_spec=grid_spec,
        out_shape=jax.ShapeDtypeStruct((PADDED, D), jnp.float32),
    )(te, tv, xs, W1, b1, W2, b2, proj_W, proj_b2d)


def _combine_body(pos_hbm, ys_hbm, out_hbm, pos_v, rows_v, sem):
    wid = lax.axis_index("s") * NC + lax.axis_index("c")
    base = wid * CHUNK
    pltpu.sync_copy(pos_hbm.at[pl.ds(base, CHUNK)], pos_v)
    pltpu.async_copy(ys_hbm.at[pos_v], rows_v, sem).wait()
    pltpu.sync_copy(rows_v, out_hbm.at[pl.ds(base, CHUNK)])


def _combine_call(pos, ys):
    mesh = plsc.VectorSubcoreMesh(core_axis_name="c", subcore_axis_name="s")
    fn = functools.partial(
        pl.kernel,
        mesh=mesh,
        out_type=jax.ShapeDtypeStruct((T, D), jnp.float32),
        scratch_types=[
            pltpu.VMEM((CHUNK,), jnp.int32),
            pltpu.VMEM((CHUNK, D), jnp.float32),
            pltpu.SemaphoreType.DMA,
        ],
    )(_combine_body)
    return fn(pos, ys)


def kernel(x, gate_W, gate_b, W1, b1, W2, b2, proj_W, proj_b):
    bs, seq_len, d_model = x.shape
    xf = x.reshape(T, D)
    gw_p = jnp.zeros((D, LANES), jnp.float32).at[:, :E].set(gate_W)
    gb_p = jnp.zeros((1, LANES), jnp.float32).at[0, :E].set(gate_b)

    pos3, counts_o, loss_o = _gate_call(xf, gw_p, gb_p)
    pos = pos3.reshape(T)
    counts = counts_o[0, :E]

    padded = ((counts + BLK - 1) // BLK) * BLK
    offs = jnp.concatenate(
        [jnp.zeros((1,), jnp.int32), jnp.cumsum(padded).astype(jnp.int32)])
    total = offs[E]
    jB = jnp.arange(MAX_TILES, dtype=jnp.int32) * BLK
    te = (jnp.sum((offs[None, :E] <= jB[:, None]).astype(jnp.int32),
                  axis=1) - 1).astype(jnp.int32)
    tv = (jB < total).astype(jnp.int32)

    xs = _dispatch_call(pos, xf)
    ys = _ffn_call(te, tv, xs, W1, b1.reshape(E, 1, H), W2,
                   b2.reshape(E, 1, D), proj_W, proj_b.reshape(1, D))
    out = _combine_call(pos, ys)

    loss = loss_o[0, 0].reshape(())
    return out.reshape(bs, seq_len, d_model), loss
